# baseline (speedup 1.0000x reference)
import functools

import jax
import jax.numpy as jnp
from jax import lax
from jax.experimental import pallas as pl
from jax.experimental.pallas import tpu as pltpu
from jax.experimental.pallas import tpu_sc as plsc

_CONFS = [(3, 48, 8, 1, 1024), (48, 96, 8, 1, 1024), (96, 192, 12, 2, 384), (192, 384, 16, 2, 128)]
_JOINT_NUM = 21


def _elu(v):
    return jnp.where(v > 0, v, jnp.exp(v) - 1.0)


def _fps_body(px, py, pz, rx, ry, rz, *, P):
    B, N = px.shape
    x = px[...]
    y = py[...]
    z = pz[...]
    sx = x[:, 0:1]
    sy = y[:, 0:1]
    sz = z[:, 0:1]
    dist = (x - sx) ** 2 + (y - sy) ** 2 + (z - sz) ** 2
    iota = lax.broadcasted_iota(jnp.int32, (B, N), 1)
    iota128 = lax.broadcasted_iota(jnp.int32, (B, 128), 1)
    zero = jnp.zeros((), jnp.float32)
    zbuf = jnp.zeros((B, 128), jnp.float32)
    col0 = iota128 == 0
    bx = jnp.where(col0, sx, zbuf)
    by = jnp.where(col0, sy, zbuf)
    bz = jnp.where(col0, sz, zbuf)
    rx[:, 0:128] = bx
    ry[:, 0:128] = by
    rz[:, 0:128] = bz

    def step(i, state):
        d, bx, by, bz = state
        m = jnp.max(d, axis=1, keepdims=True)
        idx = jnp.min(jnp.where(d == m, iota, N), axis=1, keepdims=True)
        oh = iota == idx
        sx = jnp.sum(jnp.where(oh, x, zero), axis=1, keepdims=True)
        sy = jnp.sum(jnp.where(oh, y, zero), axis=1, keepdims=True)
        sz = jnp.sum(jnp.where(oh, z, zero), axis=1, keepdims=True)
        col = iota128 == (i % 128)
        base = pl.multiple_of((i // 128) * 128, 128)
        fresh = (i % 128) == 0
        bx = jnp.where(col, sx, jnp.where(fresh, zbuf, bx))
        by = jnp.where(col, sy, jnp.where(fresh, zbuf, by))
        bz = jnp.where(col, sz, jnp.where(fresh, zbuf, bz))
        rx[:, pl.ds(base, 128)] = bx
        ry[:, pl.ds(base, 128)] = by
        rz[:, pl.ds(base, 128)] = bz
        nd = (x - sx) ** 2 + (y - sy) ** 2 + (z - sz) ** 2
        return (jnp.minimum(d, nd), bx, by, bz)

    lax.fori_loop(1, P, step, (dist, bx, by, bz))


def _fps(px, py, pz, P):
    B, N = px.shape
    out = pl.pallas_call(
        functools.partial(_fps_body, P=P),
        out_shape=[jax.ShapeDtypeStruct((B, P), jnp.float32)] * 3,
    )(px, py, pz)
    return out


def _knn_body(ptsm, rept, nn, *, N, K, D, PB):
    b = pl.program_id(0)
    pm = ptsm[...]
    rt = rept[...].reshape(3, PB)
    pxc = pm[:, 0:1]
    pyc = pm[:, 1:2]
    pzc = pm[:, 2:3]
    rxr = rt[0:1, :]
    ryr = rt[1:2, :]
    rzr = rt[2:3, :]
    d2 = (pxc - rxr) ** 2 + (pyc - ryr) ** 2 + (pzc - rzr) ** 2
    iota = lax.broadcasted_iota(jnp.int32, (N, PB), 0)
    boff = b * N
    for kk in range(K * D):
        m = jnp.min(d2, axis=0, keepdims=True)
        idx = jnp.min(jnp.where(d2 == m, iota, N), axis=0, keepdims=True)
        if kk % D == 0:
            nn[0, kk // D, :] = (idx + boff).reshape(PB)
        d2 = jnp.where(iota == idx, jnp.inf, d2)


def _knn(ptsm, rept, N, P, K, D):
    B = rept.shape[0]
    PB = 128
    out = pl.pallas_call(
        functools.partial(_knn_body, N=N, K=K, D=D, PB=PB),
        grid=(B, P // PB),
        in_specs=[
            pl.BlockSpec((N, 3), lambda b, p: (b, 0)),
            pl.BlockSpec((1, 3, PB), lambda b, p: (b, 0, p)),
        ],
        out_specs=pl.BlockSpec((1, 16, PB), lambda b, p: (b, 0, p)),
        out_shape=jax.ShapeDtypeStruct((B, 16, P), jnp.int32),
    )(ptsm, rept)
    return out


def _sc_gather(table, idx2, G):
    NCH, _ = idx2.shape
    C = table.shape[1]
    info = plsc.get_sparse_core_info()
    NC, NS = info.num_cores, info.num_subcores
    NW = NC * NS
    nch = NCH // NW
    mesh = plsc.VectorSubcoreMesh(core_axis_name="c", subcore_axis_name="s")

    @functools.partial(
        pl.kernel,
        mesh=mesh,
        out_type=jax.ShapeDtypeStruct((NW, nch, 128, C), jnp.float32),
        scratch_types=[
            pltpu.VMEM((nch, 128), jnp.int32),
            pltpu.VMEM((G, 128, C), jnp.float32),
            pltpu.SemaphoreType.DMA,
        ],
    )
    def k(tbl, idx_hbm, out_hbm, idxv, rows, sem):
        wid = lax.axis_index("s") * NC + lax.axis_index("c")
        pltpu.sync_copy(idx_hbm.at[wid], idxv)

        def group(g, carry):
            cps = []
            for j in range(G):
                cp = pltpu.async_copy(tbl.at[idxv.at[g * G + j]], rows.at[j], sem)
                cps.append(cp)
            for cp in cps:
                cp.wait()
            pltpu.sync_copy(rows, out_hbm.at[wid, pl.ds(g * G, G)])
            return carry

        lax.fori_loop(0, nch // G, group, 0)

    return k(table, idx2.reshape(NW, nch, 128))


def _xconv_body(ftsn, nbrp, repr_, nbrpf, reptl,
                d1W, d1b, d2W, d2b, t0W, t0b, t1W, t1b, t2W, t2b, eWf, eWt, eb,
                out, *, K, cin, cmid, Rb):
    ptsl = nbrp[...] - repr_[...]
    fl = _elu(jnp.dot(ptsl, d1W[...], preferred_element_type=jnp.float32) + d1b[...])
    fl = _elu(jnp.dot(fl, d2W[...], preferred_element_type=jnp.float32) + d2b[...])
    xin = nbrpf[...] - reptl[...]
    X = _elu(jnp.dot(xin, t0W[...], preferred_element_type=jnp.float32) + t0b[...])
    X = _elu(jnp.dot(X, t1W[...], preferred_element_type=jnp.float32) + t1b[...])
    X = jnp.dot(X, t2W[...], preferred_element_type=jnp.float32) + t2b[...]
    fl3 = fl.reshape(Rb, K, cmid)
    ft3 = ftsn[...].reshape(Rb, K, cin)
    flj = [fl3[:, j, :] for j in range(K)]
    ftj = [ft3[:, j, :] for j in range(K)]
    ewf = eWf[...]
    ewt = eWt[...]
    acc = jnp.zeros((Rb, eb.shape[1]), jnp.float32)
    for k in range(K):
        xc = [X[:, k * K + j : k * K + j + 1] for j in range(K)]
        afl = xc[0] * flj[0]
        aft = xc[0] * ftj[0]
        for j in range(1, K):
            afl = afl + xc[j] * flj[j]
            aft = aft + xc[j] * ftj[j]
        acc = acc + jnp.dot(afl, ewf[k * cmid:(k + 1) * cmid, :],
                            preferred_element_type=jnp.float32)
        acc = acc + jnp.dot(aft, ewt[k * cin:(k + 1) * cin, :],
                            preferred_element_type=jnp.float32)
    out[...] = _elu(acc + eb[...])


def _xconv_dense(ftsn, nbrp, repr_, nbrpf, reptl, wts, K, cin, cmid, cout, Rb):
    R = nbrpf.shape[0]
    names = ["d1W", "d1b", "d2W", "d2b", "t0W", "t0b", "t1W", "t1b", "t2W", "t2b", "eW", "eb"]
    wspecs = [pl.BlockSpec(w.shape, lambda r, nd=w.ndim: (0,) * nd) for w in wts]
    out = pl.pallas_call(
        functools.partial(_xconv_body, K=K, cin=cin, cmid=cmid, Rb=Rb),
        grid=(R // Rb,),
        in_specs=[
            pl.BlockSpec((Rb * K, cin), lambda r: (r, 0)),
            pl.BlockSpec((Rb * K, 3), lambda r: (r, 0)),
            pl.BlockSpec((Rb * K, 3), lambda r: (r, 0)),
            pl.BlockSpec((Rb, 3 * K), lambda r: (r, 0)),
            pl.BlockSpec((Rb, 3 * K), lambda r: (r, 0)),
        ] + wspecs,
        out_specs=pl.BlockSpec((Rb, cout), lambda r: (r, 0)),
        out_shape=jax.ShapeDtypeStruct((R, cout), jnp.float32),
    )(ftsn, nbrp, repr_, nbrpf, reptl, *wts)
    return out


def _head_body(fts, f1W, f1b, f2W, f2b, f3W, f3b, out, *, B, P):
    h = _elu(jnp.dot(fts[...], f1W[...], preferred_element_type=jnp.float32) + f1b[...])
    h = _elu(jnp.dot(h, f2W[...], preferred_element_type=jnp.float32) + f2b[...])
    lg = jnp.dot(h, f3W[...], preferred_element_type=jnp.float32) + f3b[...]
    lg3 = lg.reshape(B, P, lg.shape[1])
    out[...] = jnp.mean(lg3, axis=1)


def _head(fts, f1W, f1b, f2W, f2b, f3W, f3b, B, P):
    out = pl.pallas_call(
        functools.partial(_head_body, B=B, P=P),
        out_shape=jax.ShapeDtypeStruct((B, f3W.shape[1]), jnp.float32),
    )(fts, f1W, f1b, f2W, f2b, f3W, f3b)
    return out


def _ceil128(v):
    return (v + 127) // 128 * 128


_GATHER_G = {128: 4, 256: 2}


def kernel(x, params):
    B, N0, _ = x.shape
    px = x[:, :, 0]
    py = x[:, :, 1]
    pz = x[:, :, 2]
    fts = None

    for li, (cin, cout, K, D, P) in enumerate(_CONFS):
        N = px.shape[1]
        if P >= N:
            rx, ry, rz = px, py, pz
        else:
            rx, ry, rz = _fps(px, py, pz, P)

        ptsm = jnp.stack([px, py, pz], axis=-1).reshape(B * N, 3)
        rept = jnp.stack([rx, ry, rz], axis=1)
        nn = _knn(ptsm, rept, N, P, K, D)
        idx_flat = nn[:, :K, :].transpose(0, 2, 1).reshape(-1)

        ftsrows = ptsm if li == 0 else fts.reshape(B * N, cin)
        Cpad = _ceil128(cin + 3)
        pad = Cpad - (cin + 3)
        table = jnp.concatenate(
            [ftsrows, ptsm] + ([jnp.zeros((B * N, pad), jnp.float32)] if pad else []),
            axis=1)
        R = B * P * K
        gath = _sc_gather(table, idx_flat.reshape(R // 128, 128), _GATHER_G[Cpad])
        gath = gath.reshape(R, Cpad)

        ftsn = gath[:, :cin]
        nbrp = gath[:, cin:cin + 3]
        rep = jnp.stack([rx, ry, rz], axis=-1).reshape(B * P, 3)
        repr_ = jnp.broadcast_to(rep.reshape(B * P, 1, 3), (B * P, K, 3)).reshape(R, 3)
        nbrpf = nbrp.reshape(B * P, 3 * K)
        reptl = jnp.tile(rep, (1, K))
        cmid = cout // 4
        wts = []
        for n in ["d1", "d2", "t0", "t1", "t2"]:
            wts.append(params["l%d_%s_W" % (li, n)])
            wts.append(params["l%d_%s_b" % (li, n)].reshape(1, -1))
        eW3 = params["l%d_end_W" % li].reshape(K, cmid + cin, cout)
        wts.append(eW3[:, :cmid, :].reshape(K * cmid, cout))
        wts.append(eW3[:, cmid:, :].reshape(K * cin, cout))
        wts.append(params["l%d_end_b" % li].reshape(1, -1))
        Rb = 256 if K == 8 else 128
        fts = _xconv_dense(ftsn, nbrp, repr_, nbrpf, reptl, wts, K, cin, cmid, cout, Rb)
        px, py, pz = rx, ry, rz

    P = _CONFS[-1][4]
    m = _head(fts,
              params["f1_W"], params["f1_b"].reshape(1, -1),
              params["f2_W"], params["f2_b"].reshape(1, -1),
              params["f3_W"], params["f3_b"].reshape(1, -1),
              B, P)
    return m.reshape(B, _JOINT_NUM, 3)


# speedup vs baseline: 1.1300x; 1.1300x over previous
import functools

import jax
import jax.numpy as jnp
from jax import lax
from jax.experimental import pallas as pl
from jax.experimental.pallas import tpu as pltpu
from jax.experimental.pallas import tpu_sc as plsc

_CONFS = [(3, 48, 8, 1, 1024), (48, 96, 8, 1, 1024), (96, 192, 12, 2, 384), (192, 384, 16, 2, 128)]
_JOINT_NUM = 21


def _elu(v):
    return jnp.where(v > 0, v, jnp.exp(v) - 1.0)


def _fps_body(px, py, pz, rx, ry, rz, *, P):
    B, N = px.shape
    x = px[...]
    y = py[...]
    z = pz[...]
    sx = x[:, 0:1]
    sy = y[:, 0:1]
    sz = z[:, 0:1]
    dist = (x - sx) ** 2 + (y - sy) ** 2 + (z - sz) ** 2
    iota = lax.broadcasted_iota(jnp.int32, (B, N), 1)
    iota128 = lax.broadcasted_iota(jnp.int32, (B, 128), 1)
    zero = jnp.zeros((), jnp.float32)
    zbuf = jnp.zeros((B, 128), jnp.float32)
    col0 = iota128 == 0
    bx = jnp.where(col0, sx, zbuf)
    by = jnp.where(col0, sy, zbuf)
    bz = jnp.where(col0, sz, zbuf)
    rx[:, 0:128] = bx
    ry[:, 0:128] = by
    rz[:, 0:128] = bz

    def step(i, state):
        d, bx, by, bz = state
        m = jnp.max(d, axis=1, keepdims=True)
        idx = jnp.min(jnp.where(d == m, iota, N), axis=1, keepdims=True)
        oh = iota == idx
        sx = jnp.sum(jnp.where(oh, x, zero), axis=1, keepdims=True)
        sy = jnp.sum(jnp.where(oh, y, zero), axis=1, keepdims=True)
        sz = jnp.sum(jnp.where(oh, z, zero), axis=1, keepdims=True)
        col = iota128 == (i % 128)
        base = pl.multiple_of((i // 128) * 128, 128)
        fresh = (i % 128) == 0
        bx = jnp.where(col, sx, jnp.where(fresh, zbuf, bx))
        by = jnp.where(col, sy, jnp.where(fresh, zbuf, by))
        bz = jnp.where(col, sz, jnp.where(fresh, zbuf, bz))
        rx[:, pl.ds(base, 128)] = bx
        ry[:, pl.ds(base, 128)] = by
        rz[:, pl.ds(base, 128)] = bz
        nd = (x - sx) ** 2 + (y - sy) ** 2 + (z - sz) ** 2
        return (jnp.minimum(d, nd), bx, by, bz)

    lax.fori_loop(1, P, step, (dist, bx, by, bz))


def _fps(px, py, pz, P):
    B, N = px.shape
    out = pl.pallas_call(
        functools.partial(_fps_body, P=P),
        out_shape=[jax.ShapeDtypeStruct((B, P), jnp.float32)] * 3,
    )(px, py, pz)
    return out


def _knn_body(ptsm, rept, nn, *, N, K, D, PB):
    b = pl.program_id(0)
    pm = ptsm[...]
    rt = rept[...].reshape(3, PB)
    pxc = pm[:, 0:1]
    pyc = pm[:, 1:2]
    pzc = pm[:, 2:3]
    rxr = rt[0:1, :]
    ryr = rt[1:2, :]
    rzr = rt[2:3, :]
    d2 = (pxc - rxr) ** 2 + (pyc - ryr) ** 2 + (pzc - rzr) ** 2
    iota = lax.broadcasted_iota(jnp.int32, (N, PB), 0)
    boff = b * N
    for kk in range(K * D):
        m = jnp.min(d2, axis=0, keepdims=True)
        idx = jnp.min(jnp.where(d2 == m, iota, N), axis=0, keepdims=True)
        if kk % D == 0:
            nn[0, kk // D, :] = (idx + boff).reshape(PB)
        d2 = jnp.where(iota == idx, jnp.inf, d2)


def _knn(ptsm, rept, N, P, K, D):
    B = rept.shape[0]
    PB = 128
    out = pl.pallas_call(
        functools.partial(_knn_body, N=N, K=K, D=D, PB=PB),
        grid=(B, P // PB),
        in_specs=[
            pl.BlockSpec((N, 3), lambda b, p: (b, 0)),
            pl.BlockSpec((1, 3, PB), lambda b, p: (b, 0, p)),
        ],
        out_specs=pl.BlockSpec((1, 16, PB), lambda b, p: (b, 0, p)),
        out_shape=jax.ShapeDtypeStruct((B, 16, P), jnp.int32),
    )(ptsm, rept)
    return out


def _sc_gather(table, idx2, G):
    NCH, _ = idx2.shape
    C = table.shape[1]
    info = plsc.get_sparse_core_info()
    NC, NS = info.num_cores, info.num_subcores
    NW = NC * NS
    nch = NCH // NW
    mesh = plsc.VectorSubcoreMesh(core_axis_name="c", subcore_axis_name="s")

    @functools.partial(
        pl.kernel,
        mesh=mesh,
        out_type=jax.ShapeDtypeStruct((NW, nch, 128, C), jnp.float32),
        scratch_types=[
            pltpu.VMEM((nch, 128), jnp.int32),
            pltpu.VMEM((G, 128, C), jnp.float32),
            pltpu.SemaphoreType.DMA,
        ],
    )
    def k(tbl, idx_hbm, out_hbm, idxv, rows, sem):
        wid = lax.axis_index("s") * NC + lax.axis_index("c")
        pltpu.sync_copy(idx_hbm.at[wid], idxv)

        def group(g, carry):
            cps = []
            for j in range(G):
                cp = pltpu.async_copy(tbl.at[idxv.at[g * G + j]], rows.at[j], sem)
                cps.append(cp)
            for cp in cps:
                cp.wait()
            pltpu.sync_copy(rows, out_hbm.at[wid, pl.ds(g * G, G)])
            return carry

        lax.fori_loop(0, nch // G, group, 0)

    return k(table, idx2.reshape(NW, nch, 128))


def _xconv_body(gath, repm,
                d1W, d1b, d2W, d2b, t0W, t0b, t1W, t1b, t2W, t2b, eWf, eWt, eb,
                out, *, K, cin, cmid, cout, opad, Rb):
    g = gath[...]
    rep = repm[...]
    rep_rows = jnp.broadcast_to(rep.reshape(Rb, 1, 3), (Rb, K, 3)).reshape(Rb * K, 3)
    ptsl = g[:, cin:cin + 3] - rep_rows
    fl = _elu(jnp.dot(ptsl, d1W[...], preferred_element_type=jnp.float32) + d1b[...])
    fl = _elu(jnp.dot(fl, d2W[...], preferred_element_type=jnp.float32) + d2b[...])
    ptsl3 = ptsl.reshape(Rb, K, 3)
    t0w = t0W[...]
    xa = jnp.dot(ptsl3[:, 0, :], t0w[0:3, :], preferred_element_type=jnp.float32)
    for k in range(1, K):
        xa = xa + jnp.dot(ptsl3[:, k, :], t0w[3 * k:3 * k + 3, :],
                          preferred_element_type=jnp.float32)
    X = _elu(xa + t0b[...])
    X = _elu(jnp.dot(X, t1W[...], preferred_element_type=jnp.float32) + t1b[...])
    X = jnp.dot(X, t2W[...], preferred_element_type=jnp.float32) + t2b[...]
    fl3 = fl.reshape(Rb, K, cmid)
    g3 = g.reshape(Rb, K, g.shape[1])
    flj = [fl3[:, j, :] for j in range(K)]
    ftj = [g3[:, j, :cin] for j in range(K)]
    ewf = eWf[...]
    ewt = eWt[...]
    acc = jnp.zeros((Rb, cout), jnp.float32)
    for k in range(K):
        xc = [X[:, k * K + j : k * K + j + 1] for j in range(K)]
        afl = xc[0] * flj[0]
        aft = xc[0] * ftj[0]
        for j in range(1, K):
            afl = afl + xc[j] * flj[j]
            aft = aft + xc[j] * ftj[j]
        acc = acc + jnp.dot(afl, ewf[k * cmid:(k + 1) * cmid, :],
                            preferred_element_type=jnp.float32)
        acc = acc + jnp.dot(aft, ewt[k * cin:(k + 1) * cin, :],
                            preferred_element_type=jnp.float32)
    res = _elu(acc + eb[...])
    if opad:
        out[...] = jnp.concatenate(
            [res, rep, jnp.zeros((Rb, opad - cout - 3), jnp.float32)], axis=1)
    else:
        out[...] = res


def _xconv_dense(gath, repm, wts, K, cin, cmid, cout, opad, Rb):
    R = repm.shape[0]
    Cpad = gath.shape[1]
    wspecs = [pl.BlockSpec(w.shape, lambda r, nd=w.ndim: (0,) * nd) for w in wts]
    ow = opad if opad else cout
    out = pl.pallas_call(
        functools.partial(_xconv_body, K=K, cin=cin, cmid=cmid, cout=cout,
                          opad=opad, Rb=Rb),
        grid=(R // Rb,),
        in_specs=[
            pl.BlockSpec((Rb * K, Cpad), lambda r: (r, 0)),
            pl.BlockSpec((Rb, 3), lambda r: (r, 0)),
        ] + wspecs,
        out_specs=pl.BlockSpec((Rb, ow), lambda r: (r, 0)),
        out_shape=jax.ShapeDtypeStruct((R, ow), jnp.float32),
    )(gath, repm, *wts)
    return out


def _head_body(fts, f1W, f1b, f2W, f2b, f3W, f3b, out, *, B, P):
    h = _elu(jnp.dot(fts[...], f1W[...], preferred_element_type=jnp.float32) + f1b[...])
    h = _elu(jnp.dot(h, f2W[...], preferred_element_type=jnp.float32) + f2b[...])
    lg = jnp.dot(h, f3W[...], preferred_element_type=jnp.float32) + f3b[...]
    lg3 = lg.reshape(B, P, lg.shape[1])
    out[...] = jnp.mean(lg3, axis=1)


def _head(fts, f1W, f1b, f2W, f2b, f3W, f3b, B, P):
    out = pl.pallas_call(
        functools.partial(_head_body, B=B, P=P),
        out_shape=jax.ShapeDtypeStruct((B, f3W.shape[1]), jnp.float32),
    )(fts, f1W, f1b, f2W, f2b, f3W, f3b)
    return out


def _ceil128(v):
    return (v + 127) // 128 * 128


_GATHER_G = {128: 4, 256: 2}


def kernel(x, params):
    B, N0, _ = x.shape
    px = x[:, :, 0]
    py = x[:, :, 1]
    pz = x[:, :, 2]
    fts = None

    for li, (cin, cout, K, D, P) in enumerate(_CONFS):
        N = px.shape[1]
        if P >= N:
            rx, ry, rz = px, py, pz
        else:
            rx, ry, rz = _fps(px, py, pz, P)

        ptsm = jnp.stack([px, py, pz], axis=-1).reshape(B * N, 3)
        rept = jnp.stack([rx, ry, rz], axis=1)
        nn = _knn(ptsm, rept, N, P, K, D)
        idx_flat = nn[:, :K, :].transpose(0, 2, 1).reshape(-1)

        if li == 0:
            Cpad = _ceil128(cin + 3)
            table = jnp.concatenate(
                [ptsm, ptsm, jnp.zeros((B * N, Cpad - 6), jnp.float32)], axis=1)
        else:
            table = fts
            Cpad = table.shape[1]
        R = B * P * K
        gath = _sc_gather(table, idx_flat.reshape(R // 128, 128), _GATHER_G[Cpad])
        gath = gath.reshape(R, Cpad)

        rep = jnp.stack([rx, ry, rz], axis=-1).reshape(B * P, 3)
        cmid = cout // 4
        opad = 0 if li == 3 else _ceil128(cout + 3)
        wts = []
        for n in ["d1", "d2", "t0", "t1", "t2"]:
            wts.append(params["l%d_%s_W" % (li, n)])
            wts.append(params["l%d_%s_b" % (li, n)].reshape(1, -1))
        eW3 = params["l%d_end_W" % li].reshape(K, cmid + cin, cout)
        wts.append(eW3[:, :cmid, :].reshape(K * cmid, cout))
        wts.append(eW3[:, cmid:, :].reshape(K * cin, cout))
        wts.append(params["l%d_end_b" % li].reshape(1, -1))
        Rb = 256 if K == 8 else 128
        fts = _xconv_dense(gath, rep, wts, K, cin, cmid, cout, opad, Rb)
        px, py, pz = rx, ry, rz

    P = _CONFS[-1][4]
    m = _head(fts,
              params["f1_W"], params["f1_b"].reshape(1, -1),
              params["f2_W"], params["f2_b"].reshape(1, -1),
              params["f3_W"], params["f3_b"].reshape(1, -1),
              B, P)
    return m.reshape(B, _JOINT_NUM, 3)


# speedup vs baseline: 1.2852x; 1.1373x over previous
import functools

import jax
import jax.numpy as jnp
from jax import lax
from jax.experimental import pallas as pl
from jax.experimental.pallas import tpu as pltpu
from jax.experimental.pallas import tpu_sc as plsc

_CONFS = [(3, 48, 8, 1, 1024), (48, 96, 8, 1, 1024), (96, 192, 12, 2, 384), (192, 384, 16, 2, 128)]
_JOINT_NUM = 21


def _elu(v):
    return jnp.where(v > 0, v, jnp.exp(v) - 1.0)


def _fps_body(px, py, pz, rx, ry, rz, *, P):
    B, N = px.shape
    x = px[...]
    y = py[...]
    z = pz[...]
    sx = x[:, 0:1]
    sy = y[:, 0:1]
    sz = z[:, 0:1]
    dist = (x - sx) ** 2 + (y - sy) ** 2 + (z - sz) ** 2
    iota = lax.broadcasted_iota(jnp.int32, (B, N), 1)
    iota128 = lax.broadcasted_iota(jnp.int32, (B, 128), 1)
    zero = jnp.zeros((), jnp.float32)
    zbuf = jnp.zeros((B, 128), jnp.float32)
    col0 = iota128 == 0
    bx = jnp.where(col0, sx, zbuf)
    by = jnp.where(col0, sy, zbuf)
    bz = jnp.where(col0, sz, zbuf)
    rx[:, 0:128] = bx
    ry[:, 0:128] = by
    rz[:, 0:128] = bz

    def step(i, state):
        d, bx, by, bz = state
        m = jnp.max(d, axis=1, keepdims=True)
        idx = jnp.min(jnp.where(d == m, iota, N), axis=1, keepdims=True)
        oh = iota == idx
        sx = jnp.sum(jnp.where(oh, x, zero), axis=1, keepdims=True)
        sy = jnp.sum(jnp.where(oh, y, zero), axis=1, keepdims=True)
        sz = jnp.sum(jnp.where(oh, z, zero), axis=1, keepdims=True)
        col = iota128 == (i % 128)
        base = pl.multiple_of((i // 128) * 128, 128)
        fresh = (i % 128) == 0
        bx = jnp.where(col, sx, jnp.where(fresh, zbuf, bx))
        by = jnp.where(col, sy, jnp.where(fresh, zbuf, by))
        bz = jnp.where(col, sz, jnp.where(fresh, zbuf, bz))
        rx[:, pl.ds(base, 128)] = bx
        ry[:, pl.ds(base, 128)] = by
        rz[:, pl.ds(base, 128)] = bz
        nd = (x - sx) ** 2 + (y - sy) ** 2 + (z - sz) ** 2
        return (jnp.minimum(d, nd), bx, by, bz)

    lax.fori_loop(1, P, step, (dist, bx, by, bz))


def _fps(px, py, pz, P):
    B, N = px.shape
    out = pl.pallas_call(
        functools.partial(_fps_body, P=P),
        out_shape=[jax.ShapeDtypeStruct((B, P), jnp.float32)] * 3,
    )(px, py, pz)
    return out


def _knn_body(ptsm, rept, nn, *, N, K, D, PB, CH):
    b = pl.program_id(0)
    pm = ptsm[...]
    rt = rept[...].reshape(3, PB)
    rxr = rt[0:1, :]
    ryr = rt[1:2, :]
    rzr = rt[2:3, :]
    nc = N // CH
    rowi = lax.broadcasted_iota(jnp.int32, (CH, PB), 0)
    zid = jnp.zeros((CH, PB), jnp.int32)
    chunks = []
    for c in range(nc):
        pc = pm[c * CH:(c + 1) * CH]
        chunks.append((pc[:, 0:1] - rxr) ** 2 + (pc[:, 1:2] - ryr) ** 2
                      + (pc[:, 2:3] - rzr) ** 2)
    boff = b * N
    inf = jnp.float32(jnp.inf)
    idx = None
    for kk in range(K * D):
        accv = None
        for c in range(nc):
            ch = chunks[c]
            if idx is not None:
                ch = jnp.where((rowi + (c * CH)) == idx, inf, ch)
                chunks[c] = ch
            if accv is None:
                accv, accid = ch, zid
            else:
                lt = ch < accv
                accv = jnp.where(lt, ch, accv)
                accid = jnp.where(lt, jnp.int32(c), accid)
        tv = accv
        tn = accid * CH + rowi
        rows = CH
        while rows > 1:
            h = rows // 2
            av, bv = tv[:h], tv[h:]
            an, bn = tn[:h], tn[h:]
            takeb = (bv < av) | ((bv == av) & (bn < an))
            tv = jnp.where(takeb, bv, av)
            tn = jnp.where(takeb, bn, an)
            rows = h
        idx = tn
        if kk % D == 0:
            nn[0, kk // D, :] = (idx + boff).reshape(PB)


def _knn(ptsm, rept, N, P, K, D):
    B = rept.shape[0]
    PB = 128
    CH = 256 if N % 256 == 0 else 128
    out = pl.pallas_call(
        functools.partial(_knn_body, N=N, K=K, D=D, PB=PB, CH=CH),
        grid=(B, P // PB),
        in_specs=[
            pl.BlockSpec((N, 3), lambda b, p: (b, 0)),
            pl.BlockSpec((1, 3, PB), lambda b, p: (b, 0, p)),
        ],
        out_specs=pl.BlockSpec((1, 16, PB), lambda b, p: (b, 0, p)),
        out_shape=jax.ShapeDtypeStruct((B, 16, P), jnp.int32),
    )(ptsm, rept)
    return out


def _sc_gather(table, idx2, G):
    NCH, _ = idx2.shape
    C = table.shape[1]
    info = plsc.get_sparse_core_info()
    NC, NS = info.num_cores, info.num_subcores
    NW = NC * NS
    nch = NCH // NW
    mesh = plsc.VectorSubcoreMesh(core_axis_name="c", subcore_axis_name="s")

    @functools.partial(
        pl.kernel,
        mesh=mesh,
        out_type=jax.ShapeDtypeStruct((NW, nch, 128, C), jnp.float32),
        scratch_types=[
            pltpu.VMEM((nch, 128), jnp.int32),
            pltpu.VMEM((G, 128, C), jnp.float32),
            pltpu.SemaphoreType.DMA,
        ],
    )
    def k(tbl, idx_hbm, out_hbm, idxv, rows, sem):
        wid = lax.axis_index("s") * NC + lax.axis_index("c")
        pltpu.sync_copy(idx_hbm.at[wid], idxv)

        def group(g, carry):
            cps = []
            for j in range(G):
                cp = pltpu.async_copy(tbl.at[idxv.at[g * G + j]], rows.at[j], sem)
                cps.append(cp)
            for cp in cps:
                cp.wait()
            pltpu.sync_copy(rows, out_hbm.at[wid, pl.ds(g * G, G)])
            return carry

        lax.fori_loop(0, nch // G, group, 0)

    return k(table, idx2.reshape(NW, nch, 128))


def _xconv_body(gath, repm,
                d1W, d1b, d2W, d2b, t0W, t0b, t1W, t1b, t2W, t2b, eWf, eWt, eb,
                out, *, K, cin, cmid, cout, opad, Rb):
    g = gath[...]
    rep = repm[...]
    rep_rows = jnp.broadcast_to(rep.reshape(Rb, 1, 3), (Rb, K, 3)).reshape(Rb * K, 3)
    ptsl = g[:, cin:cin + 3] - rep_rows
    fl = _elu(jnp.dot(ptsl, d1W[...], preferred_element_type=jnp.float32) + d1b[...])
    fl = _elu(jnp.dot(fl, d2W[...], preferred_element_type=jnp.float32) + d2b[...])
    ptsl3 = ptsl.reshape(Rb, K, 3)
    t0w = t0W[...]
    xa = jnp.dot(ptsl3[:, 0, :], t0w[0:3, :], preferred_element_type=jnp.float32)
    for k in range(1, K):
        xa = xa + jnp.dot(ptsl3[:, k, :], t0w[3 * k:3 * k + 3, :],
                          preferred_element_type=jnp.float32)
    X = _elu(xa + t0b[...])
    X = _elu(jnp.dot(X, t1W[...], preferred_element_type=jnp.float32) + t1b[...])
    X = jnp.dot(X, t2W[...], preferred_element_type=jnp.float32) + t2b[...]
    fl3 = fl.reshape(Rb, K, cmid)
    g3 = g.reshape(Rb, K, g.shape[1])
    flj = [fl3[:, j, :] for j in range(K)]
    ftj = [g3[:, j, :cin] for j in range(K)]
    ewf = eWf[...]
    ewt = eWt[...]
    acc = jnp.zeros((Rb, cout), jnp.float32)
    for k in range(K):
        xc = [X[:, k * K + j : k * K + j + 1] for j in range(K)]
        afl = xc[0] * flj[0]
        aft = xc[0] * ftj[0]
        for j in range(1, K):
            afl = afl + xc[j] * flj[j]
            aft = aft + xc[j] * ftj[j]
        acc = acc + jnp.dot(afl, ewf[k * cmid:(k + 1) * cmid, :],
                            preferred_element_type=jnp.float32)
        acc = acc + jnp.dot(aft, ewt[k * cin:(k + 1) * cin, :],
                            preferred_element_type=jnp.float32)
    res = _elu(acc + eb[...])
    if opad:
        out[...] = jnp.concatenate(
            [res, rep, jnp.zeros((Rb, opad - cout - 3), jnp.float32)], axis=1)
    else:
        out[...] = res


def _xconv_dense(gath, repm, wts, K, cin, cmid, cout, opad, Rb):
    R = repm.shape[0]
    Cpad = gath.shape[1]
    wspecs = [pl.BlockSpec(w.shape, lambda r, nd=w.ndim: (0,) * nd) for w in wts]
    ow = opad if opad else cout
    out = pl.pallas_call(
        functools.partial(_xconv_body, K=K, cin=cin, cmid=cmid, cout=cout,
                          opad=opad, Rb=Rb),
        grid=(R // Rb,),
        in_specs=[
            pl.BlockSpec((Rb * K, Cpad), lambda r: (r, 0)),
            pl.BlockSpec((Rb, 3), lambda r: (r, 0)),
        ] + wspecs,
        out_specs=pl.BlockSpec((Rb, ow), lambda r: (r, 0)),
        out_shape=jax.ShapeDtypeStruct((R, ow), jnp.float32),
    )(gath, repm, *wts)
    return out


def _head_body(fts, f1W, f1b, f2W, f2b, f3W, f3b, out, *, B, P):
    h = _elu(jnp.dot(fts[...], f1W[...], preferred_element_type=jnp.float32) + f1b[...])
    h = _elu(jnp.dot(h, f2W[...], preferred_element_type=jnp.float32) + f2b[...])
    lg = jnp.dot(h, f3W[...], preferred_element_type=jnp.float32) + f3b[...]
    lg3 = lg.reshape(B, P, lg.shape[1])
    out[...] = jnp.mean(lg3, axis=1)


def _head(fts, f1W, f1b, f2W, f2b, f3W, f3b, B, P):
    out = pl.pallas_call(
        functools.partial(_head_body, B=B, P=P),
        out_shape=jax.ShapeDtypeStruct((B, f3W.shape[1]), jnp.float32),
    )(fts, f1W, f1b, f2W, f2b, f3W, f3b)
    return out


def _ceil128(v):
    return (v + 127) // 128 * 128


_GATHER_G = {128: 4, 256: 2}


def kernel(x, params):
    B, N0, _ = x.shape
    px = x[:, :, 0]
    py = x[:, :, 1]
    pz = x[:, :, 2]
    fts = None

    for li, (cin, cout, K, D, P) in enumerate(_CONFS):
        N = px.shape[1]
        if P >= N:
            rx, ry, rz = px, py, pz
        else:
            rx, ry, rz = _fps(px, py, pz, P)

        ptsm = jnp.stack([px, py, pz], axis=-1).reshape(B * N, 3)
        rept = jnp.stack([rx, ry, rz], axis=1)
        nn = _knn(ptsm, rept, N, P, K, D)
        idx_flat = nn[:, :K, :].transpose(0, 2, 1).reshape(-1)

        if li == 0:
            Cpad = _ceil128(cin + 3)
            table = jnp.concatenate(
                [ptsm, ptsm, jnp.zeros((B * N, Cpad - 6), jnp.float32)], axis=1)
        else:
            table = fts
            Cpad = table.shape[1]
        R = B * P * K
        gath = _sc_gather(table, idx_flat.reshape(R // 128, 128), _GATHER_G[Cpad])
        gath = gath.reshape(R, Cpad)

        rep = jnp.stack([rx, ry, rz], axis=-1).reshape(B * P, 3)
        cmid = cout // 4
        opad = 0 if li == 3 else _ceil128(cout + 3)
        wts = []
        for n in ["d1", "d2", "t0", "t1", "t2"]:
            wts.append(params["l%d_%s_W" % (li, n)])
            wts.append(params["l%d_%s_b" % (li, n)].reshape(1, -1))
        eW3 = params["l%d_end_W" % li].reshape(K, cmid + cin, cout)
        wts.append(eW3[:, :cmid, :].reshape(K * cmid, cout))
        wts.append(eW3[:, cmid:, :].reshape(K * cin, cout))
        wts.append(params["l%d_end_b" % li].reshape(1, -1))
        Rb = 256 if K == 8 else 128
        fts = _xconv_dense(gath, rep, wts, K, cin, cmid, cout, opad, Rb)
        px, py, pz = rx, ry, rz

    P = _CONFS[-1][4]
    m = _head(fts,
              params["f1_W"], params["f1_b"].reshape(1, -1),
              params["f2_W"], params["f2_b"].reshape(1, -1),
              params["f3_W"], params["f3_b"].reshape(1, -1),
              B, P)
    return m.reshape(B, _JOINT_NUM, 3)


# speedup vs baseline: 1.3433x; 1.0452x over previous
import functools

import jax
import jax.numpy as jnp
from jax import lax
from jax.experimental import pallas as pl
from jax.experimental.pallas import tpu as pltpu
from jax.experimental.pallas import tpu_sc as plsc

_CONFS = [(3, 48, 8, 1, 1024), (48, 96, 8, 1, 1024), (96, 192, 12, 2, 384), (192, 384, 16, 2, 128)]
_JOINT_NUM = 21


def _elu(v):
    return jnp.where(v > 0, v, jnp.exp(v) - 1.0)


def _fps_body(px, py, pz, rx, ry, rz, *, P):
    B, N = px.shape
    x = px[...]
    y = py[...]
    z = pz[...]
    sx = x[:, 0:1]
    sy = y[:, 0:1]
    sz = z[:, 0:1]
    dist = (x - sx) ** 2 + (y - sy) ** 2 + (z - sz) ** 2
    iota = lax.broadcasted_iota(jnp.int32, (B, N), 1)
    iota128 = lax.broadcasted_iota(jnp.int32, (B, 128), 1)
    zero = jnp.zeros((), jnp.float32)
    zbuf = jnp.zeros((B, 128), jnp.float32)
    col0 = iota128 == 0
    bx = jnp.where(col0, sx, zbuf)
    by = jnp.where(col0, sy, zbuf)
    bz = jnp.where(col0, sz, zbuf)
    rx[:, 0:128] = bx
    ry[:, 0:128] = by
    rz[:, 0:128] = bz

    def step(i, state):
        d, bx, by, bz = state
        m = jnp.max(d, axis=1, keepdims=True)
        idx = jnp.min(jnp.where(d == m, iota, N), axis=1, keepdims=True)
        oh = iota == idx
        sx = jnp.sum(jnp.where(oh, x, zero), axis=1, keepdims=True)
        sy = jnp.sum(jnp.where(oh, y, zero), axis=1, keepdims=True)
        sz = jnp.sum(jnp.where(oh, z, zero), axis=1, keepdims=True)
        col = iota128 == (i % 128)
        base = pl.multiple_of((i // 128) * 128, 128)
        fresh = (i % 128) == 0
        bx = jnp.where(col, sx, jnp.where(fresh, zbuf, bx))
        by = jnp.where(col, sy, jnp.where(fresh, zbuf, by))
        bz = jnp.where(col, sz, jnp.where(fresh, zbuf, bz))
        rx[:, pl.ds(base, 128)] = bx
        ry[:, pl.ds(base, 128)] = by
        rz[:, pl.ds(base, 128)] = bz
        nd = (x - sx) ** 2 + (y - sy) ** 2 + (z - sz) ** 2
        return (jnp.minimum(d, nd), bx, by, bz)

    lax.fori_loop(1, P, step, (dist, bx, by, bz))


def _fps(px, py, pz, P):
    B, N = px.shape
    out = pl.pallas_call(
        functools.partial(_fps_body, P=P),
        out_shape=[jax.ShapeDtypeStruct((B, P), jnp.float32)] * 3,
    )(px, py, pz)
    return out


def _knn_body(ptsm, rept, nn, *, N, K, D, PB, CH):
    b = pl.program_id(0)
    pm = ptsm[...]
    rt = rept[...].reshape(3, PB)
    rxr = rt[0:1, :]
    ryr = rt[1:2, :]
    rzr = rt[2:3, :]
    nc = N // CH
    rowi = lax.broadcasted_iota(jnp.int32, (CH, PB), 0)
    zid = jnp.zeros((CH, PB), jnp.int32)
    chunks = []
    for c in range(nc):
        pc = pm[c * CH:(c + 1) * CH]
        chunks.append((pc[:, 0:1] - rxr) ** 2 + (pc[:, 1:2] - ryr) ** 2
                      + (pc[:, 2:3] - rzr) ** 2)
    boff = b * N
    inf = jnp.float32(jnp.inf)
    idx = None
    for kk in range(K * D):
        accv = None
        for c in range(nc):
            ch = chunks[c]
            if idx is not None:
                ch = jnp.where((rowi + (c * CH)) == idx, inf, ch)
                chunks[c] = ch
            if accv is None:
                accv, accid = ch, zid
            else:
                lt = ch < accv
                accv = jnp.where(lt, ch, accv)
                accid = jnp.where(lt, jnp.int32(c), accid)
        tv = accv
        tn = accid * CH + rowi
        rows = CH
        while rows > 1:
            h = rows // 2
            av, bv = tv[:h], tv[h:]
            an, bn = tn[:h], tn[h:]
            takeb = (bv < av) | ((bv == av) & (bn < an))
            tv = jnp.where(takeb, bv, av)
            tn = jnp.where(takeb, bn, an)
            rows = h
        idx = tn
        if kk % D == 0:
            nn[0, kk // D, :] = (idx + boff).reshape(PB)


def _knn(ptsm, rept, N, P, K, D):
    B = rept.shape[0]
    PB = 128
    CH = 256 if N % 256 == 0 else 128
    out = pl.pallas_call(
        functools.partial(_knn_body, N=N, K=K, D=D, PB=PB, CH=CH),
        grid=(B, P // PB),
        in_specs=[
            pl.BlockSpec((N, 3), lambda b, p: (b, 0)),
            pl.BlockSpec((1, 3, PB), lambda b, p: (b, 0, p)),
        ],
        out_specs=pl.BlockSpec((1, 16, PB), lambda b, p: (b, 0, p)),
        out_shape=jax.ShapeDtypeStruct((B, 16, P), jnp.int32),
    )(ptsm, rept)
    return out


def _sc_gather(table, idx2, G):
    NCH, _ = idx2.shape
    C = table.shape[1]
    info = plsc.get_sparse_core_info()
    NC, NS = info.num_cores, info.num_subcores
    NW = NC * NS
    nch = NCH // NW
    mesh = plsc.VectorSubcoreMesh(core_axis_name="c", subcore_axis_name="s")

    @functools.partial(
        pl.kernel,
        mesh=mesh,
        out_type=jax.ShapeDtypeStruct((NW, nch, 128, C), jnp.float32),
        scratch_types=[
            pltpu.VMEM((nch, 128), jnp.int32),
            pltpu.VMEM((G, 128, C), jnp.float32),
            pltpu.SemaphoreType.DMA,
        ],
    )
    def k(tbl, idx_hbm, out_hbm, idxv, rows, sem):
        wid = lax.axis_index("s") * NC + lax.axis_index("c")
        pltpu.sync_copy(idx_hbm.at[wid], idxv)

        def group(g, carry):
            cps = []
            for j in range(G):
                cp = pltpu.async_copy(tbl.at[idxv.at[g * G + j]], rows.at[j], sem)
                cps.append(cp)
            for cp in cps:
                cp.wait()
            pltpu.sync_copy(rows, out_hbm.at[wid, pl.ds(g * G, G)])
            return carry

        lax.fori_loop(0, nch // G, group, 0)

    return k(table, idx2.reshape(NW, nch, 128))


def _xconv_body(gath, repm,
                d1W, d1b, d2W, d2b, t0W, t0b, t1W, t1b, t2W, t2b, eWf, eWt, eb,
                out, *, K, cin, cmid, cout, opad, Rb):
    g = gath[...]
    rep = repm[...]
    rep_rows = jnp.broadcast_to(rep.reshape(Rb, 1, 3), (Rb, K, 3)).reshape(Rb * K, 3)
    ptsl = g[:, cin:cin + 3] - rep_rows
    fl = _elu(jnp.dot(ptsl, d1W[...], preferred_element_type=jnp.float32) + d1b[...])
    fl = _elu(jnp.dot(fl, d2W[...], preferred_element_type=jnp.float32) + d2b[...])
    ptsl3 = ptsl.reshape(Rb, K, 3)
    t0w = t0W[...]
    xa = jnp.dot(ptsl3[:, 0, :], t0w[0:3, :], preferred_element_type=jnp.float32)
    for k in range(1, K):
        xa = xa + jnp.dot(ptsl3[:, k, :], t0w[3 * k:3 * k + 3, :],
                          preferred_element_type=jnp.float32)
    X = _elu(xa + t0b[...])
    X = _elu(jnp.dot(X, t1W[...], preferred_element_type=jnp.float32) + t1b[...])
    X = jnp.dot(X, t2W[...], preferred_element_type=jnp.float32) + t2b[...]
    Cc = cmid + cin
    acc = jnp.zeros((Rb, cout), jnp.float32)
    if Cc <= 128:
        fcat = jnp.concatenate([fl, g[:, :cin]], axis=1)
        fc3 = fcat.reshape(Rb, K, Cc)
        fcj = [fc3[:, j, :] for j in range(K)]
        ew = eWf[...]
        for k in range(K):
            xc = [X[:, k * K + j : k * K + j + 1] for j in range(K)]
            a = xc[0] * fcj[0]
            for j in range(1, K):
                a = a + xc[j] * fcj[j]
            acc = acc + jnp.dot(a, ew[k * Cc:(k + 1) * Cc, :],
                                preferred_element_type=jnp.float32)
    else:
        fl3 = fl.reshape(Rb, K, cmid)
        g3 = g.reshape(Rb, K, g.shape[1])
        flj = [fl3[:, j, :] for j in range(K)]
        ftj = [g3[:, j, :cin] for j in range(K)]
        ewf = eWf[...]
        ewt = eWt[...]
        for k in range(K):
            xc = [X[:, k * K + j : k * K + j + 1] for j in range(K)]
            afl = xc[0] * flj[0]
            aft = xc[0] * ftj[0]
            for j in range(1, K):
                afl = afl + xc[j] * flj[j]
                aft = aft + xc[j] * ftj[j]
            acc = acc + jnp.dot(afl, ewf[k * cmid:(k + 1) * cmid, :],
                                preferred_element_type=jnp.float32)
            acc = acc + jnp.dot(aft, ewt[k * cin:(k + 1) * cin, :],
                                preferred_element_type=jnp.float32)
    res = _elu(acc + eb[...])
    if opad:
        out[...] = jnp.concatenate(
            [res, rep, jnp.zeros((Rb, opad - cout - 3), jnp.float32)], axis=1)
    else:
        out[...] = res


def _xconv_dense(gath, repm, wts, K, cin, cmid, cout, opad, Rb):
    R = repm.shape[0]
    Cpad = gath.shape[1]
    wspecs = [pl.BlockSpec(w.shape, lambda r, nd=w.ndim: (0,) * nd) for w in wts]
    ow = opad if opad else cout
    out = pl.pallas_call(
        functools.partial(_xconv_body, K=K, cin=cin, cmid=cmid, cout=cout,
                          opad=opad, Rb=Rb),
        grid=(R // Rb,),
        in_specs=[
            pl.BlockSpec((Rb * K, Cpad), lambda r: (r, 0)),
            pl.BlockSpec((Rb, 3), lambda r: (r, 0)),
        ] + wspecs,
        out_specs=pl.BlockSpec((Rb, ow), lambda r: (r, 0)),
        out_shape=jax.ShapeDtypeStruct((R, ow), jnp.float32),
    )(gath, repm, *wts)
    return out


def _head_body(fts, f1W, f1b, f2W, f2b, f3W, f3b, out, *, B, P):
    h = _elu(jnp.dot(fts[...], f1W[...], preferred_element_type=jnp.float32) + f1b[...])
    h = _elu(jnp.dot(h, f2W[...], preferred_element_type=jnp.float32) + f2b[...])
    lg = jnp.dot(h, f3W[...], preferred_element_type=jnp.float32) + f3b[...]
    lg3 = lg.reshape(B, P, lg.shape[1])
    out[...] = jnp.mean(lg3, axis=1)


def _head(fts, f1W, f1b, f2W, f2b, f3W, f3b, B, P):
    out = pl.pallas_call(
        functools.partial(_head_body, B=B, P=P),
        out_shape=jax.ShapeDtypeStruct((B, f3W.shape[1]), jnp.float32),
    )(fts, f1W, f1b, f2W, f2b, f3W, f3b)
    return out


def _ceil128(v):
    return (v + 127) // 128 * 128


_GATHER_G = {128: 4, 256: 2}


def kernel(x, params):
    B, N0, _ = x.shape
    px = x[:, :, 0]
    py = x[:, :, 1]
    pz = x[:, :, 2]
    fts = None

    for li, (cin, cout, K, D, P) in enumerate(_CONFS):
        N = px.shape[1]
        if P >= N:
            rx, ry, rz = px, py, pz
        else:
            rx, ry, rz = _fps(px, py, pz, P)

        ptsm = jnp.stack([px, py, pz], axis=-1).reshape(B * N, 3)
        rept = jnp.stack([rx, ry, rz], axis=1)
        nn = _knn(ptsm, rept, N, P, K, D)
        idx_flat = nn[:, :K, :].transpose(0, 2, 1).reshape(-1)

        if li == 0:
            Cpad = _ceil128(cin + 3)
            table = jnp.concatenate(
                [ptsm, ptsm, jnp.zeros((B * N, Cpad - 6), jnp.float32)], axis=1)
        else:
            table = fts
            Cpad = table.shape[1]
        R = B * P * K
        gath = _sc_gather(table, idx_flat.reshape(R // 128, 128), _GATHER_G[Cpad])
        gath = gath.reshape(R, Cpad)

        rep = jnp.stack([rx, ry, rz], axis=-1).reshape(B * P, 3)
        cmid = cout // 4
        opad = 0 if li == 3 else _ceil128(cout + 3)
        wts = []
        for n in ["d1", "d2", "t0", "t1", "t2"]:
            wts.append(params["l%d_%s_W" % (li, n)])
            wts.append(params["l%d_%s_b" % (li, n)].reshape(1, -1))
        eb2 = params["l%d_end_b" % li].reshape(1, -1)
        if cmid + cin <= 128:
            wts.append(params["l%d_end_W" % li])
            wts.append(eb2)
        else:
            eW3 = params["l%d_end_W" % li].reshape(K, cmid + cin, cout)
            wts.append(eW3[:, :cmid, :].reshape(K * cmid, cout))
            wts.append(eW3[:, cmid:, :].reshape(K * cin, cout))
        wts.append(eb2)
        Rb = 512 if K == 8 else 128
        fts = _xconv_dense(gath, rep, wts, K, cin, cmid, cout, opad, Rb)
        px, py, pz = rx, ry, rz

    P = _CONFS[-1][4]
    m = _head(fts,
              params["f1_W"], params["f1_b"].reshape(1, -1),
              params["f2_W"], params["f2_b"].reshape(1, -1),
              params["f3_W"], params["f3_b"].reshape(1, -1),
              B, P)
    return m.reshape(B, _JOINT_NUM, 3)
